# baseline (speedup 1.0000x reference)
import functools

import jax
import jax.numpy as jnp
from jax import lax
from jax.experimental import pallas as pl
from jax.experimental.pallas import tpu as pltpu
from jax.experimental.pallas import tpu_sc as plsc

D = 1000
N = 1024 * 200
NW = 32
PER_W = N // NW
CHUNK = 80
STEPS = PER_W // CHUNK

_mesh = plsc.VectorSubcoreMesh(core_axis_name="c", subcore_axis_name="s")


@functools.partial(
    pl.kernel,
    mesh=_mesh,
    compiler_params=pltpu.CompilerParams(use_tc_tiling_on_sc=False),
    out_type=jax.ShapeDtypeStruct((N, D), jnp.float32),
    scratch_types=[
        pltpu.VMEM((PER_W,), jnp.int32),
        pltpu.VMEM((CHUNK, D), jnp.float32),
        pltpu.SemaphoreType.DMA,
    ],
)
def _gather(idx_hbm, table_hbm, out_hbm, idx_v, rows_v, sem):
    wid = lax.axis_index("s") * 2 + lax.axis_index("c")
    base = wid * PER_W
    pltpu.sync_copy(idx_hbm.at[pl.ds(base, PER_W)], idx_v)

    def body(i, carry):
        off = i * CHUNK
        pltpu.async_copy(
            table_hbm.at[idx_v.at[pl.ds(off, CHUNK)]], rows_v, sem
        ).wait()
        pltpu.sync_copy(rows_v, out_hbm.at[pl.ds(base + off, CHUNK)])
        return carry

    lax.fori_loop(0, STEPS, body, 0)


def kernel(idx, table):
    B, T = idx.shape
    out = _gather(idx.reshape(-1), table)
    return out.reshape(B, T, D)


# speedup vs baseline: 1.0083x; 1.0083x over previous
import functools

import jax
import jax.numpy as jnp
from jax import lax
from jax.experimental import pallas as pl
from jax.experimental.pallas import tpu as pltpu
from jax.experimental.pallas import tpu_sc as plsc

D = 1000
N = 1024 * 200
NW = 32
PER_W = N // NW
CHUNK = 40
STEPS = PER_W // CHUNK

_mesh = plsc.VectorSubcoreMesh(core_axis_name="c", subcore_axis_name="s")


@functools.partial(
    pl.kernel,
    mesh=_mesh,
    compiler_params=pltpu.CompilerParams(use_tc_tiling_on_sc=False),
    out_type=jax.ShapeDtypeStruct((N, D), jnp.float32),
    scratch_types=[
        pltpu.VMEM((PER_W,), jnp.int32),
        pltpu.VMEM((CHUNK, D), jnp.float32),
        pltpu.VMEM((CHUNK, D), jnp.float32),
        pltpu.SemaphoreType.DMA,
        pltpu.SemaphoreType.DMA,
        pltpu.SemaphoreType.DMA,
        pltpu.SemaphoreType.DMA,
    ],
)
def _gather(idx_hbm, table_hbm, out_hbm, idx_v, buf0, buf1,
            sg0, sg1, so0, so1):
    wid = lax.axis_index("s") * 2 + lax.axis_index("c")
    base = wid * PER_W
    pltpu.sync_copy(idx_hbm.at[pl.ds(base, PER_W)], idx_v)

    bufs, sg, so = (buf0, buf1), (sg0, sg1), (so0, so1)

    def gather(i, b):
        pltpu.async_copy(
            table_hbm.at[idx_v.at[pl.ds(i * CHUNK, CHUNK)]], bufs[b], sg[b]
        ).wait()

    def start_out(i, b):
        pltpu.make_async_copy(
            bufs[b], out_hbm.at[pl.ds(base + i * CHUNK, CHUNK)], so[b]
        ).start()

    def wait_out(b):
        pltpu.make_async_copy(
            bufs[b], out_hbm.at[pl.ds(base, CHUNK)], so[b]
        ).wait()

    for b in range(2):
        gather(b, b)
        start_out(b, b)

    def body(t, carry):
        for b in range(2):
            i = t * 2 + b
            wait_out(b)
            gather(i, b)
            start_out(i, b)
        return carry

    lax.fori_loop(1, STEPS // 2, body, 0)

    for b in range(2):
        wait_out(b)


def kernel(idx, table):
    B, T = idx.shape
    out = _gather(idx.reshape(-1), table)
    return out.reshape(B, T, D)


# speedup vs baseline: 1.1375x; 1.1282x over previous
import functools

import jax
import jax.numpy as jnp
from jax import lax
from jax.experimental import pallas as pl
from jax.experimental.pallas import tpu as pltpu
from jax.experimental.pallas import tpu_sc as plsc

V = 1000
D = 1000
N = 1024 * 200
NW = 32
PER_W = N // NW
CHUNK = 32
STEPS = PER_W // CHUNK

_mesh = plsc.VectorSubcoreMesh(core_axis_name="c", subcore_axis_name="s")


@functools.partial(
    pl.kernel,
    mesh=_mesh,
    compiler_params=pltpu.CompilerParams(use_tc_tiling_on_sc=False),
    out_type=jax.ShapeDtypeStruct((N, D), jnp.float32),
    scratch_types=[
        pltpu.VMEM_SHARED((V, D), jnp.float32),
        pltpu.VMEM((CHUNK,), jnp.int32),
        pltpu.VMEM((CHUNK, D), jnp.float32),
        pltpu.VMEM((CHUNK, D), jnp.float32),
        pltpu.SemaphoreType.DMA,
        pltpu.SemaphoreType.DMA,
        pltpu.SemaphoreType.DMA,
    ],
)
def _gather(idx_hbm, table_hbm, out_hbm, table_sp, idx_v, buf0, buf1,
            sg, so0, so1):
    sid = lax.axis_index("s")
    wid = sid * 2 + lax.axis_index("c")
    base = wid * PER_W

    @pl.when(sid == 0)
    def _():
        pltpu.sync_copy(table_hbm, table_sp)

    plsc.subcore_barrier()

    bufs, so = (buf0, buf1), (so0, so1)

    def gather(i, b):
        pltpu.sync_copy(idx_hbm.at[pl.ds(base + i * CHUNK, CHUNK)], idx_v)
        pltpu.async_copy(table_sp.at[idx_v], bufs[b], sg).wait()

    def start_out(i, b):
        pltpu.make_async_copy(
            bufs[b], out_hbm.at[pl.ds(base + i * CHUNK, CHUNK)], so[b]
        ).start()

    def wait_out(b):
        pltpu.make_async_copy(
            bufs[b], out_hbm.at[pl.ds(base, CHUNK)], so[b]
        ).wait()

    for b in range(2):
        gather(b, b)
        start_out(b, b)

    def body(t, carry):
        for b in range(2):
            i = t * 2 + b
            wait_out(b)
            gather(i, b)
            start_out(i, b)
        return carry

    lax.fori_loop(1, STEPS // 2, body, 0)

    for b in range(2):
        wait_out(b)


def kernel(idx, table):
    B, T = idx.shape
    out = _gather(idx.reshape(-1), table)
    return out.reshape(B, T, D)


# speedup vs baseline: 1.4486x; 1.2735x over previous
import functools

import jax
import jax.numpy as jnp
from jax import lax
from jax.experimental import pallas as pl
from jax.experimental.pallas import tpu as pltpu
from jax.experimental.pallas import tpu_sc as plsc

V = 1000
D = 1000
DM = 896
DT = 128
REP = 8
N = 1024 * 200
NW = 32
PER_W = N // NW
CHUNK = 32
STEPS = PER_W // CHUNK

_mesh = plsc.VectorSubcoreMesh(core_axis_name="c", subcore_axis_name="s")


@functools.partial(
    pl.kernel,
    mesh=_mesh,
    compiler_params=pltpu.CompilerParams(use_tc_tiling_on_sc=True),
    out_type=(
        jax.ShapeDtypeStruct((N, D), jnp.float32),
        jax.ShapeDtypeStruct((N, DT), jnp.float32),
    ),
    scratch_types=[
        pltpu.VMEM((CHUNK,), jnp.int32),
        pltpu.VMEM((CHUNK,), jnp.int32),
        pltpu.VMEM((CHUNK, DM), jnp.float32),
        pltpu.VMEM((CHUNK, DM), jnp.float32),
        pltpu.VMEM((CHUNK, DT), jnp.float32),
        pltpu.VMEM((CHUNK, DT), jnp.float32),
        pltpu.SemaphoreType.DMA,
        pltpu.SemaphoreType.DMA,
        pltpu.SemaphoreType.DMA,
        pltpu.SemaphoreType.DMA,
        pltpu.SemaphoreType.DMA,
    ],
)
def _gather(idx_hbm, tm_hbm, tt_hbm, out_hbm, tlo_hbm,
            idx_v, idx2_v, g0, g1, h0, h1,
            sg, so0, so1, st0, st1):
    sid = lax.axis_index("s")
    wid = sid * 2 + lax.axis_index("c")
    base = wid * PER_W
    roff = lax.rem(wid, REP) * V

    gb, hb, so, st = (g0, g1), (h0, h1), (so0, so1), (st0, st1)

    def gather(i, b):
        pltpu.sync_copy(idx_hbm.at[pl.ds(base + i * CHUNK, CHUNK)], idx_v)
        for j in range(CHUNK // 16):
            sl = pl.ds(j * 16, 16)
            idx2_v[sl] = idx_v[sl] + roff
        pltpu.async_copy(tm_hbm.at[idx2_v], gb[b], sg).wait()
        pltpu.async_copy(tt_hbm.at[idx2_v], hb[b], sg).wait()

    def start_out(i, b):
        pltpu.make_async_copy(
            gb[b], out_hbm.at[pl.ds(base + i * CHUNK, CHUNK), pl.ds(0, DM)],
            so[b],
        ).start()
        pltpu.make_async_copy(
            hb[b], tlo_hbm.at[pl.ds(base + i * CHUNK, CHUNK)], st[b]
        ).start()

    def wait_out(b):
        pltpu.make_async_copy(
            gb[b], out_hbm.at[pl.ds(base, CHUNK), pl.ds(0, DM)], so[b]
        ).wait()
        pltpu.make_async_copy(
            hb[b], tlo_hbm.at[pl.ds(base, CHUNK)], st[b]
        ).wait()

    for b in range(2):
        gather(b, b)
        start_out(b, b)

    def body(t, carry):
        for b in range(2):
            i = t * 2 + b
            wait_out(b)
            gather(i, b)
            start_out(i, b)
        return carry

    lax.fori_loop(1, STEPS // 2, body, 0)

    for b in range(2):
        wait_out(b)


def kernel(idx, table):
    B, T = idx.shape
    tm = jnp.tile(table[:, :DM], (REP, 1))
    tt = jnp.tile(table[:, D - DT:], (REP, 1))
    out, tail = _gather(idx.reshape(-1), tm, tt)
    out = lax.dynamic_update_slice(out, tail[:, DT - (D - DM):], (0, DM))
    return out.reshape(B, T, D)
